# Initial kernel scaffold; baseline (speedup 1.0000x reference)
import jax
import jax.numpy as jnp
from jax.experimental import pallas as pl


def kernel(user, item, user_factors, product_factors, user_bias, product_bias):
    raise NotImplementedError("write your pallas kernel here")


# speedup vs baseline: 7.4997x; 7.4997x over previous
import functools

import jax
import jax.numpy as jnp
from jax import lax
from jax.experimental import pallas as pl
from jax.experimental.pallas import tpu as pltpu
from jax.experimental.pallas import tpu_sc as plsc


def _build(total, chunk):
    info = plsc.get_sparse_core_info()
    nw = info.num_cores * info.num_subcores
    b_per_w = total // nw
    n_chunks = b_per_w // chunk
    assert b_per_w * nw == total and n_chunks * chunk == b_per_w

    mesh = plsc.VectorSubcoreMesh(core_axis_name="c", subcore_axis_name="s")

    @functools.partial(
        pl.kernel,
        mesh=mesh,
        out_type=jax.ShapeDtypeStruct((total,), jnp.float32),
        compiler_params=pltpu.CompilerParams(
            needs_layout_passes=False, use_tc_tiling_on_sc=False
        ),
        scratch_types=[
            pltpu.VMEM((chunk,), jnp.int32),
            pltpu.VMEM((chunk,), jnp.int32),
            pltpu.VMEM((chunk, 16), jnp.float32),
            pltpu.VMEM((chunk, 16), jnp.float32),
            pltpu.VMEM((chunk,), jnp.float32),
            pltpu.VMEM((chunk,), jnp.float32),
            pltpu.VMEM((chunk,), jnp.float32),
            pltpu.SemaphoreType.DMA,
            pltpu.SemaphoreType.DMA,
            pltpu.SemaphoreType.DMA,
            pltpu.SemaphoreType.DMA,
        ],
    )
    def fused_lookup(user_hbm, item_hbm, uf_hbm, pf_hbm, ub_hbm, pb_hbm,
                     out_hbm, idx_u, idx_p, urows, prows, ubv, pbv, outv,
                     sem_u, sem_p, sem_ub, sem_pb):
        wid = lax.axis_index("s") * info.num_cores + lax.axis_index("c")
        base = wid * b_per_w
        lane = lax.iota(jnp.int32, 16)

        def chunk_body(c, _):
            cbase = base + c * chunk
            pltpu.sync_copy(user_hbm.at[pl.ds(cbase, chunk)], idx_u)
            pltpu.sync_copy(item_hbm.at[pl.ds(cbase, chunk)], idx_p)
            cu = pltpu.async_copy(uf_hbm.at[idx_u], urows, sem_u)
            cp = pltpu.async_copy(pf_hbm.at[idx_p], prows, sem_p)
            cb = pltpu.async_copy(ub_hbm.at[idx_u], ubv, sem_ub)
            cq = pltpu.async_copy(pb_hbm.at[idx_p], pbv, sem_pb)
            cu.wait()
            cp.wait()
            cb.wait()
            cq.wait()

            cols = [(lane + k) & 15 for k in range(16)]

            def group_body(g, _):
                g16 = g * 16
                row = g16 + lane
                acc = ubv[pl.ds(g16, 16)] + pbv[pl.ds(g16, 16)]
                for k in range(16):
                    uc = plsc.load_gather(urows, [row, cols[k]])
                    pc = plsc.load_gather(prows, [row, cols[k]])
                    acc = acc + uc * pc
                outv[pl.ds(g16, 16)] = acc
                return 0

            lax.fori_loop(0, chunk // 16, group_body, 0)
            pltpu.sync_copy(outv, out_hbm.at[pl.ds(cbase, chunk)])
            return 0

        lax.fori_loop(0, n_chunks, chunk_body, 0)

    return fused_lookup


def kernel(user, item, user_factors, product_factors, user_bias, product_bias):
    b, l = user.shape
    total = b * l
    fused = _build(total, 2048)
    out = fused(
        user.reshape(total),
        item.reshape(total),
        user_factors,
        product_factors,
        user_bias.reshape(-1),
        product_bias.reshape(-1),
    )
    return out.reshape(b, l)
